# Initial kernel scaffold; baseline (speedup 1.0000x reference)
import jax
import jax.numpy as jnp
from jax.experimental import pallas as pl


def kernel(x, edge_index, batch, W1, b1, W2, b2, Wl, bl):
    raise NotImplementedError("write your pallas kernel here")


# speedup vs baseline: 11.7124x; 11.7124x over previous
import functools

import jax
import jax.numpy as jnp
from jax import lax
from jax.experimental import pallas as pl
from jax.experimental.pallas import tpu as pltpu
from jax.experimental.pallas import tpu_sc as plsc

N = 10000
NP = 10240
E = 320000
D = 128
G = 64

NC = 2
NS = 16
NW = NC * NS
RPS = NP // NS
CH = 80
EW = E // NW
NCH = EW // CH

@functools.cache
def _mesh():
    return plsc.VectorSubcoreMesh(core_axis_name="c", subcore_axis_name="s")


def _deg_body(dst_hbm, zeros_hbm, ones_hbm, out_hbm,
              didx_v, ones_v, acc_sh):
    c = lax.axis_index("c")
    s = lax.axis_index("s")
    wid = c * NS + s
    r0 = s * RPS
    pltpu.sync_copy(zeros_hbm.at[pl.ds(r0, RPS)], acc_sh.at[pl.ds(r0, RPS)])
    pltpu.sync_copy(ones_hbm, ones_v)
    plsc.subcore_barrier()

    base_row = wid * NCH

    def chunk(t, carry):
        pltpu.sync_copy(dst_hbm.at[base_row + t], didx_v)
        pltpu.sync_copy(ones_v, acc_sh.at[didx_v], add=True)
        return carry

    lax.fori_loop(0, NCH, chunk, 0)
    plsc.subcore_barrier()
    pltpu.sync_copy(acc_sh.at[pl.ds(r0, RPS)], out_hbm.at[c, pl.ds(r0, RPS)])


@functools.cache
def _deg_call():
    return functools.partial(
        pl.kernel,
        out_type=jax.ShapeDtypeStruct((NC, NP, D), jnp.float32),
        mesh=_mesh(),
        scratch_types=[
            pltpu.VMEM((CH,), jnp.int32),
            pltpu.VMEM((CH, D), jnp.float32),
            pltpu.VMEM_SHARED((NP, D), jnp.float32),
        ],
    )(_deg_body)


def _agg_body(hs_hbm, src_hbm, dst_hbm, zeros_hbm, out_hbm,
              sidx_v, didx_v, rows_v, acc_sh, gsem):
    c = lax.axis_index("c")
    s = lax.axis_index("s")
    wid = c * NS + s
    r0 = s * RPS
    pltpu.sync_copy(zeros_hbm.at[pl.ds(r0, RPS)], acc_sh.at[pl.ds(r0, RPS)])
    plsc.subcore_barrier()

    base_row = wid * NCH

    def chunk(t, carry):
        pltpu.sync_copy(src_hbm.at[base_row + t], sidx_v)
        pltpu.sync_copy(dst_hbm.at[base_row + t], didx_v)
        pltpu.async_copy(hs_hbm.at[sidx_v], rows_v, gsem).wait()
        pltpu.sync_copy(rows_v, acc_sh.at[didx_v], add=True)
        return carry

    lax.fori_loop(0, NCH, chunk, 0)
    plsc.subcore_barrier()
    pltpu.sync_copy(acc_sh.at[pl.ds(r0, RPS)], out_hbm.at[c, pl.ds(r0, RPS)])


@functools.cache
def _agg_call():
    return functools.partial(
        pl.kernel,
        out_type=jax.ShapeDtypeStruct((NC, NP, D), jnp.float32),
        mesh=_mesh(),
        scratch_types=[
            pltpu.VMEM((CH,), jnp.int32),
            pltpu.VMEM((CH,), jnp.int32),
            pltpu.VMEM((CH, D), jnp.float32),
            pltpu.VMEM_SHARED((NP, D), jnp.float32),
            pltpu.SemaphoreType.DMA,
        ],
    )(_agg_body)


_R = 1024
_GRID = NP // _R


def _scale_mm_body(x_ref, w_ref, da_ref, db_ref, o_ref, dinv_ref):
    e0 = (lax.broadcasted_iota(jnp.int32, (D, 1), 0) == 0).astype(jnp.float32)
    deg = jnp.dot(da_ref[...] + db_ref[...], e0,
                  preferred_element_type=jnp.float32) + 1.0
    dinv = lax.rsqrt(deg)
    h = jnp.dot(x_ref[...], w_ref[...], preferred_element_type=jnp.float32)
    o_ref[...] = h * dinv
    dinv_ref[...] = dinv


def _tc_hs1(xpad, w1, dega, degb):
    return pl.pallas_call(
        _scale_mm_body,
        grid=(_GRID,),
        in_specs=[
            pl.BlockSpec((_R, D), lambda k: (k, 0)),
            pl.BlockSpec((D, D), lambda k: (0, 0)),
            pl.BlockSpec((_R, D), lambda k: (k, 0)),
            pl.BlockSpec((_R, D), lambda k: (k, 0)),
        ],
        out_specs=[
            pl.BlockSpec((_R, D), lambda k: (k, 0)),
            pl.BlockSpec((_R, 1), lambda k: (k, 0)),
        ],
        out_shape=[
            jax.ShapeDtypeStruct((NP, D), jnp.float32),
            jax.ShapeDtypeStruct((NP, 1), jnp.float32),
        ],
    )(xpad, w1, dega, degb)


def _layer2_body(aa_ref, ab_ref, hs_ref, dinv_ref, w_ref, b_ref, o_ref):
    dinv = dinv_ref[...]
    z = (aa_ref[...] + ab_ref[...] + hs_ref[...]) * dinv + b_ref[...]
    z = jnp.maximum(z, 0.0)
    h = jnp.dot(z, w_ref[...], preferred_element_type=jnp.float32)
    o_ref[...] = h * dinv


def _tc_hs2(agga, aggb, hs1, dinv, w2, b1):
    return pl.pallas_call(
        _layer2_body,
        grid=(_GRID,),
        in_specs=[
            pl.BlockSpec((_R, D), lambda k: (k, 0)),
            pl.BlockSpec((_R, D), lambda k: (k, 0)),
            pl.BlockSpec((_R, D), lambda k: (k, 0)),
            pl.BlockSpec((_R, 1), lambda k: (k, 0)),
            pl.BlockSpec((D, D), lambda k: (0, 0)),
            pl.BlockSpec((1, D), lambda k: (0, 0)),
        ],
        out_specs=pl.BlockSpec((_R, D), lambda k: (k, 0)),
        out_shape=jax.ShapeDtypeStruct((NP, D), jnp.float32),
    )(agga, aggb, hs1, dinv, w2, b1)


def _final_body(aa_ref, ab_ref, hs_ref, dinv_ref, b_ref, batch_ref,
                wl_ref, bl_ref, o_ref, acc_s, acc_c):
    k = pl.program_id(0)

    @pl.when(k == 0)
    def _():
        acc_s[...] = jnp.zeros_like(acc_s)
        acc_c[...] = jnp.zeros_like(acc_c)

    dinv = dinv_ref[...]
    z = (aa_ref[...] + ab_ref[...] + hs_ref[...]) * dinv + b_ref[...]
    z = jnp.maximum(z, 0.0)
    ids = batch_ref[...]
    oht = (lax.broadcasted_iota(jnp.int32, (G, _R), 0) == ids
           ).astype(jnp.float32)
    acc_s[...] += jnp.dot(oht, z, preferred_element_type=jnp.float32)
    acc_c[...] += jnp.dot(oht, jnp.ones((_R, 1), jnp.float32),
                          preferred_element_type=jnp.float32)

    @pl.when(k == _GRID - 1)
    def _():
        pooled = acc_s[...] / jnp.maximum(acc_c[...], 1.0)
        o_ref[...] = jnp.dot(pooled, wl_ref[...],
                             preferred_element_type=jnp.float32) + bl_ref[...]


def _tc_final(agga, aggb, hs2, dinv, b2, batch_p, wl, bl):
    return pl.pallas_call(
        _final_body,
        grid=(_GRID,),
        in_specs=[
            pl.BlockSpec((_R, D), lambda k: (k, 0)),
            pl.BlockSpec((_R, D), lambda k: (k, 0)),
            pl.BlockSpec((_R, D), lambda k: (k, 0)),
            pl.BlockSpec((_R, 1), lambda k: (k, 0)),
            pl.BlockSpec((1, D), lambda k: (0, 0)),
            pl.BlockSpec((1, _R), lambda k: (0, k)),
            pl.BlockSpec((D, D), lambda k: (0, 0)),
            pl.BlockSpec((1, D), lambda k: (0, 0)),
        ],
        out_specs=pl.BlockSpec((G, D), lambda k: (0, 0)),
        out_shape=jax.ShapeDtypeStruct((G, D), jnp.float32),
        scratch_shapes=[
            pltpu.VMEM((G, D), jnp.float32),
            pltpu.VMEM((G, 1), jnp.float32),
        ],
    )(agga, aggb, hs2, dinv, b2, batch_p, wl, bl)


def kernel(x, edge_index, batch, W1, b1, W2, b2, Wl, bl):
    f32 = jnp.float32
    src = edge_index[0].astype(jnp.int32).reshape(E // CH, CH)
    dst = edge_index[1].astype(jnp.int32).reshape(E // CH, CH)
    xpad = jnp.zeros((NP, D), f32).at[:N].set(x.astype(f32))
    zeros128 = jnp.zeros((NP, D), f32)
    ones80 = jnp.ones((CH, D), f32)

    deg2 = _deg_call()(dst, zeros128, ones80)

    hs1, dinv = _tc_hs1(xpad, W1, deg2[0], deg2[1])
    agg1 = _agg_call()(hs1, src, dst, zeros128)
    hs2 = _tc_hs2(agg1[0], agg1[1], hs1, dinv, W2,
                  b1.reshape(1, D))
    agg2 = _agg_call()(hs2, src, dst, zeros128)

    batch_p = jnp.full((1, NP), G, jnp.int32).at[0, :N].set(
        batch.astype(jnp.int32))
    out = _tc_final(agg2[0], agg2[1], hs2, dinv,
                    b2.reshape(1, D), batch_p, Wl, bl.reshape(1, D))
    return out


# speedup vs baseline: 19.8572x; 1.6954x over previous
import functools

import jax
import jax.numpy as jnp
from jax import lax
from jax.experimental import pallas as pl
from jax.experimental.pallas import tpu as pltpu
from jax.experimental.pallas import tpu_sc as plsc

N = 10000
NP = 10240
E = 320000
D = 128
G = 64

NC = 2
NS = 16
NW = NC * NS
RPS = NP // NS
CH = 80
EW = E // NW
NCH = EW // CH
NB = 5
NGRP = NCH // NB

CH2 = 40
NCH2 = EW // CH2
NSTEP = NCH2 // NB

@functools.cache
def _mesh():
    return plsc.VectorSubcoreMesh(core_axis_name="c", subcore_axis_name="s")


def _deg_body(dst_hbm, zeros_hbm, ones_hbm, out_hbm,
              didx_v, ones_v, acc_sh, dsem):
    c = lax.axis_index("c")
    s = lax.axis_index("s")
    wid = c * NS + s
    r0 = s * RPS
    pltpu.sync_copy(dst_hbm.at[wid], didx_v)
    pltpu.sync_copy(zeros_hbm.at[pl.ds(r0, RPS)], acc_sh.at[pl.ds(r0, RPS)])
    pltpu.sync_copy(ones_hbm, ones_v)
    plsc.subcore_barrier()

    def fire(g):
        for b in range(NB):
            pltpu.async_copy(ones_v, acc_sh.at[didx_v.at[g * NB + b]],
                             dsem, add=True)

    def drain():
        for b in range(NB):
            pltpu.make_async_copy(ones_v, acc_sh.at[didx_v.at[0]],
                                  dsem).wait()

    fire(0)

    def grp(g, carry):
        @pl.when(g < NGRP - 1)
        def _():
            fire(g + 1)
        drain()
        return carry

    lax.fori_loop(0, NGRP, grp, 0)
    plsc.subcore_barrier()
    pltpu.sync_copy(acc_sh.at[pl.ds(r0, RPS)], out_hbm.at[c, pl.ds(r0, RPS)])


@functools.cache
def _deg_call():
    return functools.partial(
        pl.kernel,
        out_type=jax.ShapeDtypeStruct((NC, NP, D), jnp.float32),
        mesh=_mesh(),
        scratch_types=[
            pltpu.VMEM((NCH, CH), jnp.int32),
            pltpu.VMEM((CH, D), jnp.float32),
            pltpu.VMEM_SHARED((NP, D), jnp.float32),
            pltpu.SemaphoreType.DMA,
        ],
    )(_deg_body)


def _agg_body(hs_hbm, src_hbm, dst_hbm, zeros_hbm, out_hbm,
              sidx_v, didx_v, rows_v, acc_sh, gsem, ssem):
    c = lax.axis_index("c")
    s = lax.axis_index("s")
    wid = c * NS + s
    r0 = s * RPS
    pltpu.sync_copy(zeros_hbm.at[pl.ds(r0, RPS)], acc_sh.at[pl.ds(r0, RPS)])
    plsc.subcore_barrier()

    def step(g, bank):
        pltpu.sync_copy(src_hbm.at[wid, g], sidx_v.at[bank])
        pltpu.sync_copy(dst_hbm.at[wid, g], didx_v.at[bank])
        for b in range(NB):
            @pl.when(g >= 1)
            def _():
                pltpu.make_async_copy(
                    rows_v.at[b], acc_sh.at[didx_v.at[bank, b]],
                    ssem.at[b]).wait()
            pltpu.async_copy(hs_hbm.at[sidx_v.at[bank, b]],
                             rows_v.at[b], gsem.at[b])
        for b in range(NB):
            pltpu.make_async_copy(hs_hbm.at[sidx_v.at[bank, b]],
                                  rows_v.at[b], gsem.at[b]).wait()
            pltpu.async_copy(rows_v.at[b],
                             acc_sh.at[didx_v.at[bank, b]], ssem.at[b],
                             add=True)

    def pair(i, carry):
        step(2 * i, 0)
        step(2 * i + 1, 1)
        return carry

    lax.fori_loop(0, NSTEP // 2, pair, 0)

    for b in range(NB):
        pltpu.make_async_copy(rows_v.at[b], acc_sh.at[didx_v.at[0, 0]],
                              ssem.at[b]).wait()
    plsc.subcore_barrier()
    pltpu.sync_copy(acc_sh.at[pl.ds(r0, RPS)], out_hbm.at[c, pl.ds(r0, RPS)])


@functools.cache
def _agg_call():
    return functools.partial(
        pl.kernel,
        out_type=jax.ShapeDtypeStruct((NC, NP, D), jnp.float32),
        mesh=_mesh(),
        scratch_types=[
            pltpu.VMEM((2, NB, CH2), jnp.int32),
            pltpu.VMEM((2, NB, CH2), jnp.int32),
            pltpu.VMEM((NB, CH2, D), jnp.float32),
            pltpu.VMEM_SHARED((NP, D), jnp.float32),
            pltpu.SemaphoreType.DMA((NB,)),
            pltpu.SemaphoreType.DMA((NB,)),
        ],
    )(_agg_body)


_R = 1024
_GRID = NP // _R


def _scale_mm_body(x_ref, w_ref, da_ref, db_ref, o_ref, dinv_ref):
    e0 = (lax.broadcasted_iota(jnp.int32, (D, 1), 0) == 0).astype(jnp.float32)
    deg = jnp.dot(da_ref[...] + db_ref[...], e0,
                  preferred_element_type=jnp.float32) + 1.0
    dinv = lax.rsqrt(deg)
    h = jnp.dot(x_ref[...], w_ref[...], preferred_element_type=jnp.float32)
    o_ref[...] = h * dinv
    dinv_ref[...] = dinv


def _tc_hs1(xpad, w1, dega, degb):
    return pl.pallas_call(
        _scale_mm_body,
        grid=(_GRID,),
        in_specs=[
            pl.BlockSpec((_R, D), lambda k: (k, 0)),
            pl.BlockSpec((D, D), lambda k: (0, 0)),
            pl.BlockSpec((_R, D), lambda k: (k, 0)),
            pl.BlockSpec((_R, D), lambda k: (k, 0)),
        ],
        out_specs=[
            pl.BlockSpec((_R, D), lambda k: (k, 0)),
            pl.BlockSpec((_R, 1), lambda k: (k, 0)),
        ],
        out_shape=[
            jax.ShapeDtypeStruct((NP, D), jnp.float32),
            jax.ShapeDtypeStruct((NP, 1), jnp.float32),
        ],
    )(xpad, w1, dega, degb)


def _layer2_body(aa_ref, ab_ref, hs_ref, dinv_ref, w_ref, b_ref, o_ref):
    dinv = dinv_ref[...]
    z = (aa_ref[...] + ab_ref[...] + hs_ref[...]) * dinv + b_ref[...]
    z = jnp.maximum(z, 0.0)
    h = jnp.dot(z, w_ref[...], preferred_element_type=jnp.float32)
    o_ref[...] = h * dinv


def _tc_hs2(agga, aggb, hs1, dinv, w2, b1):
    return pl.pallas_call(
        _layer2_body,
        grid=(_GRID,),
        in_specs=[
            pl.BlockSpec((_R, D), lambda k: (k, 0)),
            pl.BlockSpec((_R, D), lambda k: (k, 0)),
            pl.BlockSpec((_R, D), lambda k: (k, 0)),
            pl.BlockSpec((_R, 1), lambda k: (k, 0)),
            pl.BlockSpec((D, D), lambda k: (0, 0)),
            pl.BlockSpec((1, D), lambda k: (0, 0)),
        ],
        out_specs=pl.BlockSpec((_R, D), lambda k: (k, 0)),
        out_shape=jax.ShapeDtypeStruct((NP, D), jnp.float32),
    )(agga, aggb, hs1, dinv, w2, b1)


def _final_body(aa_ref, ab_ref, hs_ref, dinv_ref, b_ref, batch_ref,
                wl_ref, bl_ref, o_ref, acc_s, acc_c):
    k = pl.program_id(0)

    @pl.when(k == 0)
    def _():
        acc_s[...] = jnp.zeros_like(acc_s)
        acc_c[...] = jnp.zeros_like(acc_c)

    dinv = dinv_ref[...]
    z = (aa_ref[...] + ab_ref[...] + hs_ref[...]) * dinv + b_ref[...]
    z = jnp.maximum(z, 0.0)
    ids = batch_ref[...]
    oht = (lax.broadcasted_iota(jnp.int32, (G, _R), 0) == ids
           ).astype(jnp.float32)
    acc_s[...] += jnp.dot(oht, z, preferred_element_type=jnp.float32)
    acc_c[...] += jnp.dot(oht, jnp.ones((_R, 1), jnp.float32),
                          preferred_element_type=jnp.float32)

    @pl.when(k == _GRID - 1)
    def _():
        pooled = acc_s[...] / jnp.maximum(acc_c[...], 1.0)
        o_ref[...] = jnp.dot(pooled, wl_ref[...],
                             preferred_element_type=jnp.float32) + bl_ref[...]


def _tc_final(agga, aggb, hs2, dinv, b2, batch_p, wl, bl):
    return pl.pallas_call(
        _final_body,
        grid=(_GRID,),
        in_specs=[
            pl.BlockSpec((_R, D), lambda k: (k, 0)),
            pl.BlockSpec((_R, D), lambda k: (k, 0)),
            pl.BlockSpec((_R, D), lambda k: (k, 0)),
            pl.BlockSpec((_R, 1), lambda k: (k, 0)),
            pl.BlockSpec((1, D), lambda k: (0, 0)),
            pl.BlockSpec((1, _R), lambda k: (0, k)),
            pl.BlockSpec((D, D), lambda k: (0, 0)),
            pl.BlockSpec((1, D), lambda k: (0, 0)),
        ],
        out_specs=pl.BlockSpec((G, D), lambda k: (0, 0)),
        out_shape=jax.ShapeDtypeStruct((G, D), jnp.float32),
        scratch_shapes=[
            pltpu.VMEM((G, D), jnp.float32),
            pltpu.VMEM((G, 1), jnp.float32),
        ],
    )(agga, aggb, hs2, dinv, b2, batch_p, wl, bl)


def kernel(x, edge_index, batch, W1, b1, W2, b2, Wl, bl):
    f32 = jnp.float32
    src = edge_index[0].astype(jnp.int32)
    dst = edge_index[1].astype(jnp.int32)
    src2 = src.reshape(NW, NSTEP, NB, CH2)
    dst2 = dst.reshape(NW, NSTEP, NB, CH2)
    dst8 = dst.reshape(NW, NCH, CH)
    xpad = jnp.zeros((NP, D), f32).at[:N].set(x.astype(f32))
    zeros128 = jnp.zeros((NP, D), f32)
    ones80 = jnp.ones((CH, D), f32)

    deg2 = _deg_call()(dst8, zeros128, ones80)

    hs1, dinv = _tc_hs1(xpad, W1, deg2[0], deg2[1])
    agg1 = _agg_call()(hs1, src2, dst2, zeros128)
    hs2 = _tc_hs2(agg1[0], agg1[1], hs1, dinv, W2,
                  b1.reshape(1, D))
    agg2 = _agg_call()(hs2, src2, dst2, zeros128)

    batch_p = jnp.full((1, NP), G, jnp.int32).at[0, :N].set(
        batch.astype(jnp.int32))
    out = _tc_final(agg2[0], agg2[1], hs2, dinv,
                    b2.reshape(1, D), batch_p, Wl, bl.reshape(1, D))
    return out
